# Initial kernel scaffold; baseline (speedup 1.0000x reference)
import jax
import jax.numpy as jnp
from jax.experimental import pallas as pl


def kernel(x, params):
    raise NotImplementedError("write your pallas kernel here")


# speedup vs baseline: 2.0644x; 2.0644x over previous
import math
import numpy as np
import jax
import jax.numpy as jnp
from jax.experimental import pallas as pl
from jax.experimental.pallas import tpu as pltpu

IN_CH = 5; P = 8; SFH = 64; EMB = 128; HID = 512; HEADS = 4; FFN = 2048
LAYERS = 2; NSPEC = 6; TOPK = 2; NSHARED = 2; NCLS = 2; ALPHA = 1e-4
B = 2; S = 256; T = B * S; DH = HID // HEADS


def _pos_encoding_np():
    pos = np.arange(S, dtype=np.float32)[:, None]
    div = np.exp(np.arange(0, EMB, 2, dtype=np.float32) * (-math.log(10000.0) / EMB))
    pe = np.zeros((S, EMB), np.float32)
    pe[:, 0::2] = np.sin(pos * div)
    pe[:, 1::2] = np.cos(pos * div)
    return np.tile(pe, (B, 1))


def _ln(x, g, b, eps=1e-5):
    m = x.mean(-1, keepdims=True)
    v = ((x - m) ** 2).mean(-1, keepdims=True)
    return (x - m) / jnp.sqrt(v + eps) * g + b


def _embed_kernel(t_ref, w1, b1, w2, b2, pos, pw, pb, out_ref):
    t = jnp.maximum(t_ref[...] @ w1[...] + b1[...], 0.0)
    t = jnp.maximum(t @ w2[...] + b2[...], 0.0)
    t = t + pos[...]
    out_ref[...] = t @ pw[...] + pb[...]


def _attn_kernel(x_ref, qkvw, qkvb, outw, outb, n1g, n1b, spr, shr,
                 y_ref, g_ref, oh_ref, rp_ref):
    bidx = pl.program_id(0)
    x = x_ref[0]
    qkv = x @ qkvw[...] + qkvb[...]
    outs = []
    scale = 1.0 / math.sqrt(DH)
    for hd in range(HEADS):
        q = qkv[:, hd * DH:(hd + 1) * DH]
        k = qkv[:, HID + hd * DH: HID + (hd + 1) * DH]
        v = qkv[:, 2 * HID + hd * DH: 2 * HID + (hd + 1) * DH]
        s = jax.lax.dot_general(q, k, (((1,), (1,)), ((), ()))) * scale
        a = jax.nn.softmax(s, axis=-1)
        outs.append(a @ v)
    o = jnp.concatenate(outs, axis=1)
    att = o @ outw[...] + outb[...]
    y = _ln(x + att, n1g[...], n1b[...])
    y_ref[0] = y

    pr = jax.nn.softmax(y @ spr[...], axis=-1)
    p1 = jnp.full((S, 1), -1.0, jnp.float32)
    i1 = jnp.zeros((S, 1), jnp.int32)
    for e in range(NSPEC):
        pe = pr[:, e:e + 1]
        upd = pe > p1
        i1 = jnp.where(upd, e, i1)
        p1 = jnp.where(upd, pe, p1)
    p2 = jnp.full((S, 1), -1.0, jnp.float32)
    i2 = jnp.zeros((S, 1), jnp.int32)
    for e in range(NSPEC):
        pe = pr[:, e:e + 1]
        upd = (pe > p2) & (i1 != e)
        i2 = jnp.where(upd, e, i2)
        p2 = jnp.where(upd, pe, p2)
    den = p1 + p2 + 1e-9
    w1n = p1 / den
    w2n = p2 / den
    spec_g = []
    for e in range(NSPEC):
        ge = jnp.where(i1 == e, w1n, 0.0) + jnp.where(i2 == e, w2n, 0.0)
        spec_g.append(ge)
    sh_p = jax.nn.softmax(y @ shr[...], axis=-1)
    g_ref[0] = jnp.concatenate(spec_g + [sh_p[:, 0:1], sh_p[:, 1:2]], axis=1)

    oh = []
    for e in range(NSPEC):
        oh.append(jnp.sum(((i1 == e) | (i2 == e)).astype(jnp.float32),
                          axis=0, keepdims=True))
    oh_row = jnp.concatenate(oh, axis=1)
    rp_row = jnp.sum(pr, axis=0, keepdims=True)

    @pl.when(bidx == 0)
    def _():
        oh_ref[...] = oh_row
        rp_ref[...] = rp_row

    @pl.when(bidx > 0)
    def _():
        oh_ref[...] += oh_row
        rp_ref[...] += rp_row


def _moe_dense_kernel(y_ref, w1_ref, b1_ref, w2_ref, b2_ref, g_ref, out_ref):
    e = pl.program_id(0)
    z = y_ref[...] @ w1_ref[0] + b1_ref[0]
    h1 = 0.5 * z * (1.0 + jax.lax.erf(z * (1.0 / math.sqrt(2.0))))
    o = (h1 @ w2_ref[0] + b2_ref[0]) * g_ref[0]

    @pl.when(e == 0)
    def _():
        out_ref[...] = o

    @pl.when(e > 0)
    def _():
        out_ref[...] += o


def _combine_kernel(y_ref, spec_ref, shar_ref, mng, mnb, n2g, n2b, out_ref):
    y = y_ref[...]
    m = _ln(y + spec_ref[...] + shar_ref[...], mng[...], mnb[...])
    out_ref[...] = _ln(y + m, n2g[...], n2b[...])


def _head_kernel(h_ref, rw, rb, cw, cb, oh_ref, rp_ref,
                 recon_ref, logits_ref, aux_ref):
    h = h_ref[...]
    recon_ref[...] = h @ rw[...] + rb[...]
    pooled = jnp.concatenate(
        [jnp.mean(h[b * S:(b + 1) * S], axis=0, keepdims=True)
         for b in range(B)], axis=0)
    logits_ref[...] = pooled @ cw[...] + cb[...]
    ohm = oh_ref[...] / float(T)
    rpm = rp_ref[...] / float(T)
    aux_ref[...] = jnp.sum(ohm * rpm).reshape(1, 1)


def _full_spec(shape):
    return pl.BlockSpec(shape, lambda *_: (0,) * len(shape))


def _run_embed(t, p):
    pos = jnp.asarray(_pos_encoding_np())
    return pl.pallas_call(
        _embed_kernel,
        out_shape=jax.ShapeDtypeStruct((T, HID), jnp.float32),
    )(t, p['sf_w1'], p['sf_b1'].reshape(1, -1), p['sf_w2'],
      p['sf_b2'].reshape(1, -1), pos, p['proj_w'], p['proj_b'].reshape(1, -1))


def _run_attn(h, L):
    y, g, oh, rp = pl.pallas_call(
        _attn_kernel,
        grid=(B,),
        in_specs=[
            pl.BlockSpec((1, S, HID), lambda b: (b, 0, 0)),
            _full_spec((HID, 3 * HID)), _full_spec((1, 3 * HID)),
            _full_spec((HID, HID)), _full_spec((1, HID)),
            _full_spec((1, HID)), _full_spec((1, HID)),
            _full_spec((HID, NSPEC)), _full_spec((HID, NSHARED)),
        ],
        out_specs=[
            pl.BlockSpec((1, S, HID), lambda b: (b, 0, 0)),
            pl.BlockSpec((1, S, NSPEC + NSHARED), lambda b: (b, 0, 0)),
            _full_spec((1, NSPEC)), _full_spec((1, NSPEC)),
        ],
        out_shape=[
            jax.ShapeDtypeStruct((B, S, HID), jnp.float32),
            jax.ShapeDtypeStruct((B, S, NSPEC + NSHARED), jnp.float32),
            jax.ShapeDtypeStruct((1, NSPEC), jnp.float32),
            jax.ShapeDtypeStruct((1, NSPEC), jnp.float32),
        ],
    )(h.reshape(B, S, HID), L['qkv_w'], L['qkv_b'].reshape(1, -1),
      L['out_w'], L['out_b'].reshape(1, -1),
      L['n1_g'].reshape(1, -1), L['n1_b'].reshape(1, -1),
      L['sp_router'], L['sh_router'])
    return y.reshape(T, HID), g.reshape(T, NSPEC + NSHARED), oh, rp


def _run_dense_experts(y, w1, b1, w2, b2, gcols):
    E = w1.shape[0]
    return pl.pallas_call(
        _moe_dense_kernel,
        grid=(E,),
        in_specs=[
            _full_spec((T, HID)),
            pl.BlockSpec((1, HID, FFN), lambda e: (e, 0, 0)),
            pl.BlockSpec((1, 1, FFN), lambda e: (e, 0, 0)),
            pl.BlockSpec((1, FFN, HID), lambda e: (e, 0, 0)),
            pl.BlockSpec((1, 1, HID), lambda e: (e, 0, 0)),
            pl.BlockSpec((1, T, 1), lambda e: (e, 0, 0)),
        ],
        out_specs=_full_spec((T, HID)),
        out_shape=jax.ShapeDtypeStruct((T, HID), jnp.float32),
    )(y, w1, b1.reshape(E, 1, -1), w2, b2.reshape(E, 1, -1), gcols)


def _run_combine(y, spec, shar, L):
    return pl.pallas_call(
        _combine_kernel,
        out_shape=jax.ShapeDtypeStruct((T, HID), jnp.float32),
    )(y, spec, shar, L['mn_g'].reshape(1, -1), L['mn_b'].reshape(1, -1),
      L['n2_g'].reshape(1, -1), L['n2_b'].reshape(1, -1))


def _run_head(h, p, oh, rp):
    return pl.pallas_call(
        _head_kernel,
        out_shape=[
            jax.ShapeDtypeStruct((T, EMB), jnp.float32),
            jax.ShapeDtypeStruct((B, NCLS), jnp.float32),
            jax.ShapeDtypeStruct((1, 1), jnp.float32),
        ],
    )(h, p['recon_w'], p['recon_b'].reshape(1, -1),
      p['cls_w'], p['cls_b'].reshape(1, -1), oh, rp)


def kernel(x, params):
    nH, nW = 128 // P, 128 // P
    t = x.reshape(B, IN_CH, nH, P, nW, P).transpose(0, 1, 2, 4, 3, 5)
    t = t.reshape(B, IN_CH, nH * nW, P, P).transpose(0, 2, 1, 3, 4)
    t = t.reshape(T, IN_CH * P * P)

    h = _run_embed(t, params)
    oh_list, rp_list = [], []
    for L in params['layers']:
        y, g, oh, rp = _run_attn(h, L)
        spec_cols = g[:, :NSPEC].T.reshape(NSPEC, T, 1)
        shar_cols = g[:, NSPEC:].T.reshape(NSHARED, T, 1)
        spec = _run_dense_experts(y, L['sp_fc1_w'], L['sp_fc1_b'],
                                  L['sp_fc2_w'], L['sp_fc2_b'], spec_cols)
        shar = _run_dense_experts(y, L['sh_fc1_w'], L['sh_fc1_b'],
                                  L['sh_fc2_w'], L['sh_fc2_b'], shar_cols)
        h = _run_combine(y, spec, shar, L)
        oh_list.append(oh)
        rp_list.append(rp)

    recon, logits, auxm = _run_head(
        h, params, jnp.concatenate(oh_list, 0), jnp.concatenate(rp_list, 0))
    aux = ALPHA * NSPEC * auxm.reshape(())
    return logits, recon.reshape(B, S, EMB), aux
